# baseline (speedup 1.0000x reference)
import functools

import jax
import jax.numpy as jnp
from jax import lax
from jax.experimental import pallas as pl

N = 10000
E = 320000
D = 128
DE = 16
EPS = 1e-5


def _tables_body(x_ref, wf_ref, ws_ref, tdst_ref, tsrc_ref):
    x = x_ref[...]
    wf = wf_ref[...]
    ws = ws_ref[...]
    dn = (((1,), (1,)), ((), ()))
    f32 = jnp.float32
    tdst_ref[:, :D] = lax.dot_general(x, wf[:, :D], dn, preferred_element_type=f32)
    tdst_ref[:, D:] = lax.dot_general(x, ws[:, :D], dn, preferred_element_type=f32)
    tsrc_ref[:, :D] = lax.dot_general(x, wf[:, D:2 * D], dn, preferred_element_type=f32)
    tsrc_ref[:, D:] = lax.dot_general(x, ws[:, D:2 * D], dn, preferred_element_type=f32)


def _node_tables(x, Wf, Ws):
    return pl.pallas_call(
        _tables_body,
        out_shape=(
            jax.ShapeDtypeStruct((N, 2 * D), jnp.float32),
            jax.ShapeDtypeStruct((N, 2 * D), jnp.float32),
        ),
    )(x, Wf, Ws)


_BE = 8000


def _epack_body(ea_ref, wfe_ref, wse_ref, bf_ref, bs_ref, out_ref):
    ea = ea_ref[...]
    dn = (((1,), (1,)), ((), ()))
    f32 = jnp.float32
    out_ref[:, :D] = lax.dot_general(ea, wfe_ref[...], dn, preferred_element_type=f32) + bf_ref[...]
    out_ref[:, D:] = lax.dot_general(ea, wse_ref[...], dn, preferred_element_type=f32) + bs_ref[...]


def _edge_tables(edge_attr, Wfe, Wse, bf, bs):
    grid = E // _BE
    return pl.pallas_call(
        _epack_body,
        grid=(grid,),
        in_specs=[
            pl.BlockSpec((_BE, DE), lambda i: (i, 0)),
            pl.BlockSpec((D, DE), lambda i: (0, 0)),
            pl.BlockSpec((D, DE), lambda i: (0, 0)),
            pl.BlockSpec((1, D), lambda i: (0, 0)),
            pl.BlockSpec((1, D), lambda i: (0, 0)),
        ],
        out_specs=pl.BlockSpec((_BE, 2 * D), lambda i: (i, 0)),
        out_shape=jax.ShapeDtypeStruct((E, 2 * D), jnp.float32),
    )(edge_attr, Wfe, Wse, bf.reshape(1, D), bs.reshape(1, D))


def _post_body(p_ref, x_ref, bng_ref, bnb_ref, lng_ref, lnb_ref, o_ref):
    agg = p_ref[0] + p_ref[1]
    x = x_ref[...]
    mean = jnp.mean(agg, axis=0, keepdims=True)
    d = agg - mean
    var = jnp.mean(d * d, axis=0, keepdims=True)
    agg_bn = d * lax.rsqrt(var + EPS) * bng_ref[...] + bnb_ref[...]
    conv = agg_bn + x
    mu = jnp.mean(conv, axis=1, keepdims=True)
    dd = conv - mu
    v = jnp.mean(dd * dd, axis=1, keepdims=True)
    h = dd * lax.rsqrt(v + EPS) * lng_ref[...] + lnb_ref[...]
    o_ref[...] = jnp.maximum(h, 0.0) + x


def _postprocess(partials, x, bn_gamma, bn_beta, ln_gamma, ln_beta):
    return pl.pallas_call(
        _post_body,
        out_shape=jax.ShapeDtypeStruct((N, D), jnp.float32),
    )(partials, x, bn_gamma.reshape(1, D), bn_beta.reshape(1, D),
      ln_gamma.reshape(1, D), ln_beta.reshape(1, D))


def _edge_phase_xla(tdst, tsrc, epack, dst, src):
    p = jnp.take(tdst, dst, axis=0) + jnp.take(tsrc, src, axis=0) + epack
    msg = jax.nn.sigmoid(p[:, :D]) * jax.nn.softplus(p[:, D:])
    agg = jax.ops.segment_sum(msg, dst, num_segments=N)
    return jnp.stack([agg, jnp.zeros_like(agg)], axis=0)


def kernel(x, edge_index, edge_attr, Wf, bf, Ws, bs, bn_gamma, bn_beta, ln_gamma, ln_beta):
    src = edge_index[0].astype(jnp.int32)
    dst = edge_index[1].astype(jnp.int32)
    tdst, tsrc = _node_tables(x, Wf, Ws)
    epack = _edge_tables(edge_attr, Wf[:, 2 * D:], Ws[:, 2 * D:], bf, bs)
    partials = _edge_phase_xla(tdst, tsrc, epack, dst, src)
    return _postprocess(partials, x, bn_gamma, bn_beta, ln_gamma, ln_beta)
